# baseline (speedup 1.0000x reference)
import functools

import jax
import jax.numpy as jnp
from jax import lax
from jax.experimental import pallas as pl
from jax.experimental.pallas import tpu as pltpu
from jax.experimental.pallas import tpu_sc as plsc

_B = 16384
_D = 16
_NC = 2
_NS = 16
_NW = _NC * _NS
_BPW = _B // _NW

_mesh = plsc.VectorSubcoreMesh(core_axis_name="c", subcore_axis_name="s")


@functools.partial(
    pl.kernel,
    mesh=_mesh,
    out_type=jax.ShapeDtypeStruct((_B, _D), jnp.float32),
    compiler_params=pltpu.CompilerParams(use_tc_tiling_on_sc=False),
    scratch_types=[
        pltpu.VMEM((_BPW,), jnp.int32),
        pltpu.VMEM((_BPW, _D), jnp.float32),
        pltpu.SemaphoreType.DMA,
    ],
)
def _embed_sc(idx_hbm, table_hbm, out_hbm, idx_v, rows_v, sem):
    wid = lax.axis_index("s") * _NC + lax.axis_index("c")
    base = wid * _BPW
    pltpu.sync_copy(idx_hbm.at[pl.ds(base, _BPW)], idx_v)
    pltpu.async_copy(table_hbm.at[idx_v], rows_v, sem).wait()
    pltpu.sync_copy(rows_v, out_hbm.at[pl.ds(base, _BPW)])


def kernel(op, table):
    return _embed_sc(op.astype(jnp.int32), table)


# speedup vs baseline: 5.3802x; 5.3802x over previous
import functools

import jax
import jax.numpy as jnp
from jax import lax
from jax.experimental import pallas as pl
from jax.experimental.pallas import tpu as pltpu
from jax.experimental.pallas import tpu_sc as plsc

_B = 16384
_D = 16
_V = 1000000
_NC = 2
_NS = 16
_NW = _NC * _NS
_BPW = _B // _NW
_K = 16
_L = 16

_mesh = plsc.VectorSubcoreMesh(core_axis_name="c", subcore_axis_name="s")


@functools.partial(
    pl.kernel,
    mesh=_mesh,
    out_type=jax.ShapeDtypeStruct((_D, _B), jnp.float32),
    compiler_params=pltpu.CompilerParams(needs_layout_passes=False),
    scratch_types=[
        pltpu.VMEM((_BPW,), jnp.int32),
        pltpu.VMEM((_K, _D, 128), jnp.float32),
        pltpu.VMEM((_D, _BPW), jnp.float32),
    ]
    + [pltpu.SemaphoreType.DMA] * _K,
)
def _embed_sc(idx_hbm, tableT_hbm, outT_hbm, idx_v, tiles_v, cols_v, *sems):
    wid = lax.axis_index("s") * _NC + lax.axis_index("c")
    base = wid * _BPW
    pltpu.sync_copy(idx_hbm.at[pl.ds(base, _BPW)], idx_v)

    rows16 = lax.iota(jnp.int32, _L)
    n_grp = _BPW // _K

    def fire(r, i):
        c = pl.multiple_of((i >> 7) * 128, 128)
        pltpu.make_async_copy(
            tableT_hbm.at[:, pl.ds(c, 128)], tiles_v.at[r], sems[r]
        ).start()

    def drain(r):
        pltpu.make_async_copy(
            tableT_hbm.at[:, pl.ds(0, 128)], tiles_v.at[r], sems[r]
        ).wait()

    chunk0 = idx_v[pl.ds(0, _K)]
    for r in range(_K):
        fire(r, chunk0[r])

    def body(g, carry):
        chunk = idx_v[pl.ds(g * _K, _K)]
        g_next = jnp.minimum(g + 1, n_grp - 1)
        chunk_next = idx_v[pl.ds(g_next * _K, _K)]
        for r in range(_K):
            drain(r)
            lane = chunk[r] & 127
            col = plsc.load_gather(
                tiles_v.at[r], [rows16, jnp.full((_L,), lane, jnp.int32)]
            )
            plsc.store_scatter(
                cols_v, [rows16, jnp.full((_L,), g * _K + r, jnp.int32)], col
            )

            @pl.when(g < n_grp - 1)
            def _():
                fire(r, chunk_next[r])

        return carry

    lax.fori_loop(0, n_grp, body, 0)
    pltpu.sync_copy(cols_v, outT_hbm.at[:, pl.ds(base, _BPW)])


def kernel(op, table):
    outT = _embed_sc(op.astype(jnp.int32), table.T)
    return outT.T
